# baseline (speedup 1.0000x reference)
import functools

import jax
import jax.numpy as jnp
from jax import lax
from jax.experimental import pallas as pl
from jax.experimental.pallas import tpu as pltpu
from jax.experimental.pallas import tpu_sc as plsc

N = 10000
E = 320000
D = 128

NC = 2
NS = 16
LANES = 16

NW = NC * NS
EDGES_PER_W = E // NW
CHUNK = 80
NCHUNKS = 128
FUSE = 8
NROW_CHUNKS = N // CHUNK

_DNUMS = lax.GatherDimensionNumbers(
    offset_dims=(), collapsed_slice_dims=(0,), start_index_map=(0,))


def _sc_body(x_hbm, idx_hbm, out_hbm, ibuf, gbuf, gbuf1, gbuf2, agg,
             gsem0, gsem1, gsem2):
    c = lax.axis_index("c")
    s = lax.axis_index("s")
    w = c * NS + s

    zero16 = jnp.zeros((LANES,), jnp.float32)

    def _zrow(r, carry):
        for k in range(D // LANES):
            gbuf[r, pl.ds(k * LANES, LANES)] = zero16
        return carry

    lax.fori_loop(0, CHUNK, _zrow, 0)

    for i in range((NROW_CHUNKS + NS - 1) // NS):
        cid = s + i * NS

        @pl.when(cid < NROW_CHUNKS)
        def _zero_chunk():
            r0 = pl.multiple_of(cid * CHUNK, CHUNK)
            pltpu.sync_copy(gbuf, agg.at[pl.ds(r0, CHUNK)])

    plsc.subcore_barrier()

    def scale(gb, vrow):
        def grp(g, carry):
            vv = plsc.bitcast(
                ibuf[vrow, pl.ds(g * LANES, LANES)], jnp.float32)
            for jj in range(LANES):
                bc = lax.gather(
                    vv, jnp.full((LANES, 1), jj, jnp.int32), _DNUMS, (1,),
                    mode=lax.GatherScatterMode.PROMISE_IN_BOUNDS)
                r = g * LANES + jj
                for k in range(D // LANES):
                    sl = pl.ds(k * LANES, LANES)
                    gb[r, sl] = gb[r, sl] * bc
            return carry

        lax.fori_loop(0, CHUNK // LANES, grp, 0)

    bufs = [(gbuf, gsem0), (gbuf1, gsem1), (gbuf2, gsem2)]

    def _super(u, carry):
        pltpu.sync_copy(idx_hbm.at[w, u], ibuf)
        pltpu.async_copy(x_hbm.at[ibuf.at[0]], gbuf, gsem0)
        for k in range(FUSE):
            gb, gs = bufs[k % 3]
            if k >= 2:
                pb, ps = bufs[(k - 2) % 3]
                pltpu.make_async_copy(
                    pb, agg.at[ibuf.at[3 * (k - 2) + 1]], ps).wait()
            if k + 1 < FUSE:
                nb, ns = bufs[(k + 1) % 3]
                pltpu.async_copy(x_hbm.at[ibuf.at[3 * (k + 1)]], nb, ns)
            pltpu.make_async_copy(x_hbm.at[ibuf.at[3 * k]], gb, gs).wait()
            scale(gb, 3 * k + 2)
            pltpu.async_copy(gb, agg.at[ibuf.at[3 * k + 1]], gs, add=True)
        for k in (FUSE - 2, FUSE - 1):
            pb, ps = bufs[k % 3]
            pltpu.make_async_copy(pb, agg.at[ibuf.at[3 * k + 1]], ps).wait()
        return carry

    lax.fori_loop(0, NCHUNKS // FUSE, _super, 0)
    plsc.subcore_barrier()

    for i in range((NROW_CHUNKS + NS - 1) // NS):
        cid = s + i * NS

        @pl.when(cid < NROW_CHUNKS)
        def _copy_chunk():
            r0 = pl.multiple_of(cid * CHUNK, CHUNK)
            pltpu.sync_copy(agg.at[pl.ds(r0, CHUNK)],
                            out_hbm.at[c, pl.ds(r0, CHUNK)])


_sc_segment_sum = functools.partial(
    pl.kernel,
    out_type=jax.ShapeDtypeStruct((NC, N, D), jnp.float32),
    mesh=plsc.VectorSubcoreMesh(core_axis_name="c", subcore_axis_name="s"),
    compiler_params=pltpu.CompilerParams(needs_layout_passes=False),
    scratch_types=[
        pltpu.VMEM((3 * FUSE, CHUNK), jnp.int32),
        pltpu.VMEM((CHUNK, D), jnp.float32),
        pltpu.VMEM((CHUNK, D), jnp.float32),
        pltpu.VMEM((CHUNK, D), jnp.float32),
        pltpu.VMEM_SHARED((N, D), jnp.float32),
        pltpu.SemaphoreType.DMA,
        pltpu.SemaphoreType.DMA,
        pltpu.SemaphoreType.DMA,
    ],
)(_sc_body)


BLK = 1000


def _tc_linear_body(p0_ref, p1_ref, wt_ref, b_ref, o_ref):
    acc = p0_ref[...] + p1_ref[...]
    o_ref[...] = (
        jnp.dot(acc, wt_ref[...], preferred_element_type=jnp.float32)
        + b_ref[...]
    )


def _tc_linear(p0, p1, wt, b2):
    return pl.pallas_call(
        _tc_linear_body,
        grid=(N // BLK,),
        in_specs=[
            pl.BlockSpec((BLK, D), lambda i: (i, 0)),
            pl.BlockSpec((BLK, D), lambda i: (i, 0)),
            pl.BlockSpec((D, D), lambda i: (0, 0)),
            pl.BlockSpec((1, D), lambda i: (0, 0)),
        ],
        out_specs=pl.BlockSpec((BLK, D), lambda i: (i, 0)),
        out_shape=jax.ShapeDtypeStruct((N, D), jnp.float32),
    )(p0, p1, wt, b2)


def kernel(X, L_rows, L_cols, L_vals, W, b):
    nsup = NCHUNKS // FUSE
    pad = NCHUNKS * CHUNK - EDGES_PER_W
    rows_p = jnp.pad(L_rows.reshape(NW, EDGES_PER_W), ((0, 0), (0, pad)))
    cols_p = jnp.pad(L_cols.reshape(NW, EDGES_PER_W), ((0, 0), (0, pad)))
    vals_p = jnp.pad(L_vals.reshape(NW, EDGES_PER_W), ((0, 0), (0, pad)))
    cols4 = cols_p.reshape(NW, nsup, FUSE, 1, CHUNK)
    rows4 = rows_p.reshape(NW, nsup, FUSE, 1, CHUNK)
    vals4 = jax.lax.bitcast_convert_type(
        vals_p.reshape(NW, nsup, FUSE, 1, CHUNK), jnp.int32)
    idx = jnp.concatenate([cols4, rows4, vals4], axis=3).reshape(
        NW, nsup, 3 * FUSE, CHUNK)
    partials = _sc_segment_sum(X, idx)
    return _tc_linear(partials[0], partials[1], W.T, b.reshape(1, D))


# speedup vs baseline: 2.2422x; 2.2422x over previous
import functools

import jax
import jax.numpy as jnp
from jax import lax
from jax.experimental import pallas as pl
from jax.experimental.pallas import tpu as pltpu
from jax.experimental.pallas import tpu_sc as plsc

N = 10000
E = 320000
D = 128

NC = 2
NS = 16
LANES = 16

NW = NC * NS
EDGES_PER_W = E // NW
CHUNK = 80
NCHUNKS = EDGES_PER_W // CHUNK
FUSE = 5
NROW_CHUNKS = N // CHUNK

_DNUMS = lax.GatherDimensionNumbers(
    offset_dims=(), collapsed_slice_dims=(0,), start_index_map=(0,))


def _sc_body(x_hbm, idx_hbm, out_hbm, ibuf, gbuf, gbuf1, gbuf2, agg,
             gsem0, gsem1, gsem2):
    c = lax.axis_index("c")
    s = lax.axis_index("s")
    w = c * NS + s

    zero16 = jnp.zeros((LANES,), jnp.float32)

    def _zrow(r, carry):
        for k in range(D // LANES):
            gbuf[r, pl.ds(k * LANES, LANES)] = zero16
        return carry

    lax.fori_loop(0, CHUNK, _zrow, 0)

    for i in range((NROW_CHUNKS + NS - 1) // NS):
        cid = s + i * NS

        @pl.when(cid < NROW_CHUNKS)
        def _zero_chunk():
            r0 = pl.multiple_of(cid * CHUNK, CHUNK)
            pltpu.sync_copy(gbuf, agg.at[pl.ds(r0, CHUNK)])

    plsc.subcore_barrier()

    def scale(gb, vrow):
        def grp(g, carry):
            vv = plsc.bitcast(
                ibuf[vrow, pl.ds(g * LANES, LANES)], jnp.float32)
            for jj in range(LANES):
                bc = lax.gather(
                    vv, jnp.full((LANES, 1), jj, jnp.int32), _DNUMS, (1,),
                    mode=lax.GatherScatterMode.PROMISE_IN_BOUNDS)
                r = g * LANES + jj
                for k in range(D // LANES):
                    sl = pl.ds(k * LANES, LANES)
                    gb[r, sl] = gb[r, sl] * bc
            return carry

        lax.fori_loop(0, CHUNK // LANES, grp, 0)

    bufs = [(gbuf, gsem0), (gbuf1, gsem1), (gbuf2, gsem2)]

    def _super(u, carry):
        pltpu.sync_copy(idx_hbm.at[w, u], ibuf)
        pltpu.async_copy(x_hbm.at[ibuf.at[0]], gbuf, gsem0)
        for k in range(FUSE):
            gb, gs = bufs[k % 3]
            if k >= 2:
                pb, ps = bufs[(k - 2) % 3]
                pltpu.make_async_copy(
                    pb, agg.at[ibuf.at[3 * (k - 2) + 1]], ps).wait()
            if k + 1 < FUSE:
                nb, ns = bufs[(k + 1) % 3]
                pltpu.async_copy(x_hbm.at[ibuf.at[3 * (k + 1)]], nb, ns)
            pltpu.make_async_copy(x_hbm.at[ibuf.at[3 * k]], gb, gs).wait()
            scale(gb, 3 * k + 2)
            pltpu.async_copy(gb, agg.at[ibuf.at[3 * k + 1]], gs, add=True)
        for k in (FUSE - 2, FUSE - 1):
            pb, ps = bufs[k % 3]
            pltpu.make_async_copy(pb, agg.at[ibuf.at[3 * k + 1]], ps).wait()
        return carry

    lax.fori_loop(0, NCHUNKS // FUSE, _super, 0)
    plsc.subcore_barrier()

    for i in range((NROW_CHUNKS + NS - 1) // NS):
        cid = s + i * NS

        @pl.when(cid < NROW_CHUNKS)
        def _copy_chunk():
            r0 = pl.multiple_of(cid * CHUNK, CHUNK)
            pltpu.sync_copy(agg.at[pl.ds(r0, CHUNK)],
                            out_hbm.at[c, pl.ds(r0, CHUNK)])


_sc_segment_sum = functools.partial(
    pl.kernel,
    out_type=jax.ShapeDtypeStruct((NC, N, D), jnp.float32),
    mesh=plsc.VectorSubcoreMesh(core_axis_name="c", subcore_axis_name="s"),
    compiler_params=pltpu.CompilerParams(needs_layout_passes=False),
    scratch_types=[
        pltpu.VMEM((3 * FUSE, CHUNK), jnp.int32),
        pltpu.VMEM((CHUNK, D), jnp.float32),
        pltpu.VMEM((CHUNK, D), jnp.float32),
        pltpu.VMEM((CHUNK, D), jnp.float32),
        pltpu.VMEM_SHARED((N, D), jnp.float32),
        pltpu.SemaphoreType.DMA,
        pltpu.SemaphoreType.DMA,
        pltpu.SemaphoreType.DMA,
    ],
)(_sc_body)


BLK = 1000


def _tc_linear_body(p0_ref, p1_ref, wt_ref, b_ref, o_ref):
    acc = p0_ref[...] + p1_ref[...]
    o_ref[...] = (
        jnp.dot(acc, wt_ref[...], preferred_element_type=jnp.float32)
        + b_ref[...]
    )


def _tc_linear(p0, p1, wt, b2):
    return pl.pallas_call(
        _tc_linear_body,
        grid=(N // BLK,),
        in_specs=[
            pl.BlockSpec((BLK, D), lambda i: (i, 0)),
            pl.BlockSpec((BLK, D), lambda i: (i, 0)),
            pl.BlockSpec((D, D), lambda i: (0, 0)),
            pl.BlockSpec((1, D), lambda i: (0, 0)),
        ],
        out_specs=pl.BlockSpec((BLK, D), lambda i: (i, 0)),
        out_shape=jax.ShapeDtypeStruct((N, D), jnp.float32),
    )(p0, p1, wt, b2)


def kernel(X, L_rows, L_cols, L_vals, W, b):
    nsup = NCHUNKS // FUSE
    cols4 = L_cols.reshape(NW, nsup, FUSE, 1, CHUNK)
    rows4 = L_rows.reshape(NW, nsup, FUSE, 1, CHUNK)
    vals4 = jax.lax.bitcast_convert_type(
        L_vals.reshape(NW, nsup, FUSE, 1, CHUNK), jnp.int32)
    idx = jnp.concatenate([cols4, rows4, vals4], axis=3).reshape(
        NW, nsup, 3 * FUSE, CHUNK)
    partials = _sc_segment_sum(X, idx)
    return _tc_linear(partials[0], partials[1], W.T, b.reshape(1, D))
